# baseline (speedup 1.0000x reference)
import functools

import jax
import jax.numpy as jnp
from jax.experimental import pallas as pl
from jax.experimental.pallas import tpu as pltpu

_K = 1024
_COMMITMENT_COST = 0.25


def _vq_chain(zb, cb):
    scores = jax.lax.dot_general(
        cb, zb, (((1,), (0,)), ((), ())),
        preferred_element_type=jnp.float32)
    c2 = jnp.sum(cb * cb, axis=1, keepdims=True)
    z2 = jnp.sum(zb * zb, axis=0, keepdims=True)
    dists = (z2 - 2.0 * scores) + c2

    minv = jnp.min(dists, axis=0, keepdims=True)
    rows = jax.lax.broadcasted_iota(jnp.int32, dists.shape, 0)
    idx = jnp.min(jnp.where(dists == minv, rows, _K),
                  axis=0, keepdims=True)
    onehot = (rows == idx).astype(jnp.float32)

    zq = jax.lax.dot_general(
        cb, onehot, (((0,), (0,)), ((), ())),
        preferred_element_type=jnp.float32)

    return zq, idx, jnp.sum(minv)


def _vq_kernel(n_split, z_ref, cb_ref, zq_ref, idx_ref, sse_ref):
    cb = cb_ref[...]
    ts = z_ref.shape[2] // n_split

    tile_sse = 0.0
    for i in range(n_split):
        sl = pl.ds(i * ts, ts)
        zq, idx, sse = _vq_chain(z_ref[0, :, sl], cb)
        zq_ref[0, :, sl] = zq
        idx_ref[0, 0, :, sl] = idx
        tile_sse += sse

    @pl.when(jnp.logical_and(pl.program_id(0) == 0, pl.program_id(1) == 0))
    def _init():
        sse_ref[0, 0] = 0.0

    sse_ref[0, 0] += tile_sse


@functools.partial(jax.jit, static_argnames=("tile_s", "n_split"))
def _vq(z, codebook, tile_s=8192, n_split=4):
    B, D, d, h, w = z.shape
    S = d * h * w
    ns = S // tile_s
    zr = z.reshape(B, D, S)

    zq, idx, sse = pl.pallas_call(
        functools.partial(_vq_kernel, n_split),
        grid=(B, ns),
        in_specs=[
            pl.BlockSpec((1, D, tile_s), lambda b, s: (b, 0, s)),
            pl.BlockSpec((_K, D), lambda b, s: (0, 0)),
        ],
        out_specs=[
            pl.BlockSpec((1, D, tile_s), lambda b, s: (b, 0, s)),
            pl.BlockSpec((1, 1, 1, tile_s), lambda b, s: (b, s, 0, 0)),
            pl.BlockSpec(memory_space=pltpu.SMEM),
        ],
        out_shape=[
            jax.ShapeDtypeStruct((B, D, S), jnp.float32),
            jax.ShapeDtypeStruct((B, ns, 1, tile_s), jnp.int32),
            jax.ShapeDtypeStruct((1, 1), jnp.float32),
        ],
    )(zr, codebook)

    loss = sse[0, 0] * (1.0 + _COMMITMENT_COST) / z.size
    return (zq.reshape(B, D, d, h, w), loss, idx.reshape(B, d, h, w))


def kernel(z, codebook):
    return _vq(z, codebook)
